# baseline (speedup 1.0000x reference)
import functools

import jax
import jax.numpy as jnp
from jax.experimental import pallas as pl
from jax.experimental.pallas import tpu as pltpu

_H = 12
_HD = 64

_CP = pltpu.CompilerParams(vmem_limit_bytes=110 * 1024 * 1024)


def _embed_body(ids_ref, emb_ref, pos_ref, o_ref):
    o_ref[...] = emb_ref[...] + pos_ref[...]


def _embed(input_ids, embedding, pos_emb):
    s = input_ids.shape[0]
    v, d = embedding.shape
    grid_spec = pltpu.PrefetchScalarGridSpec(
        num_scalar_prefetch=1,
        grid=(s,),
        in_specs=[
            pl.BlockSpec((1, 1, d), lambda i, ids: (ids[i], 0, 0)),
            pl.BlockSpec((1, 1, d), lambda i, ids: (i, 0, 0)),
        ],
        out_specs=pl.BlockSpec((1, 1, d), lambda i, ids: (i, 0, 0)),
    )
    out = pl.pallas_call(
        _embed_body,
        grid_spec=grid_spec,
        out_shape=jax.ShapeDtypeStruct((s, 1, d), jnp.float32),
        compiler_params=_CP,
    )(input_ids, embedding.reshape(v, 1, d), pos_emb.reshape(s, 1, d))
    return out.reshape(s, d)


def _mm1x(a, b, dims=(((1,), (1,)), ((), ()))):
    return jax.lax.dot_general(a.astype(jnp.bfloat16), b.astype(jnp.bfloat16),
                               dims, preferred_element_type=jnp.float32)


def _ln(x, w, b):
    m = jnp.mean(x, axis=-1, keepdims=True)
    v = jnp.mean((x - m) ** 2, axis=-1, keepdims=True)
    return (x - m) / jnp.sqrt(v + 1e-6) * w + b


def _pre_body(cap, x_ref, rw_ref, w_ref):
    x = x_ref[...]
    s = x.shape[0]
    rl = _mm1x(x, rw_ref[...])[:, 0:1]
    bits = jax.lax.bitcast_convert_type(rl, jnp.int32)
    skey = jnp.where(bits >= 0, bits, bits ^ jnp.int32(0x7FFFFFFF))
    ukey = jax.lax.bitcast_convert_type(skey ^ jnp.int32(-2147483648),
                                        jnp.uint32)

    def step(i, t):
        bit = jnp.uint32(31) - i.astype(jnp.uint32)
        cand = t | (jnp.uint32(1) << bit)
        cnt = jnp.sum((ukey >= cand).astype(jnp.float32))
        return jnp.where(cnt >= cap, cand, t)

    t = jax.lax.fori_loop(0, 32, step, jnp.uint32(0))
    gt = ukey > t
    eq = ukey == t
    n_gt = jnp.sum(gt.astype(jnp.float32))
    n_eq_needed = cap - n_gt
    rows = jax.lax.broadcasted_iota(jnp.int32, (s, s), 0)
    cols = jax.lax.broadcasted_iota(jnp.int32, (s, s), 1)
    lower = (rows >= cols).astype(jnp.float32)
    cum_eq = jnp.dot(lower, eq.astype(jnp.float32),
                     preferred_element_type=jnp.float32)
    mask = jnp.where(gt | (eq & (cum_eq <= n_eq_needed)), 1.0, 0.0)
    probs = jax.nn.sigmoid(rl)
    w_ref[...] = (mask + probs) - probs


def _pre(x, rw, cap):
    s, d = x.shape
    return pl.pallas_call(
        functools.partial(_pre_body, float(cap)),
        out_shape=jax.ShapeDtypeStruct((s, 1), jnp.float32),
        compiler_params=_CP,
    )(x, jnp.zeros((128, d), jnp.float32).at[0].set(rw))


def _qkv_body(xn_ref, qw_ref, kw_ref, vw_ref, q_ref, k_ref, v_ref):
    xn = xn_ref[...]
    q_ref[...] = _mm1x(xn, qw_ref[...])
    k_ref[...] = _mm1x(xn, kw_ref[...])
    v_ref[...] = _mm1x(xn, vw_ref[...])


def _qkv(xn, qw, kw, vw):
    s, d = xn.shape
    sds = jax.ShapeDtypeStruct((s, d), jnp.float32)
    return pl.pallas_call(
        _qkv_body,
        out_shape=(sds, sds, sds),
        compiler_params=_CP,
    )(xn, qw, kw, vw)


def _post_body(x_ref, a_ref, ow_ref, w_ref, y_ref):
    proj = _mm1x(a_ref[...], ow_ref[...])
    y_ref[...] = x_ref[...] + w_ref[...] * proj


def _post(x, attn_out, ow, wv):
    s, d = x.shape
    return pl.pallas_call(
        _post_body,
        out_shape=jax.ShapeDtypeStruct((s, d), jnp.float32),
        compiler_params=_CP,
    )(x, attn_out, ow, wv)


def _ffn_gu_body(xn_ref, gw_ref, uw_ref, g_ref, u_ref):
    xn = xn_ref[...]
    g_ref[...] = _mm1x(xn, gw_ref[...])
    u_ref[...] = _mm1x(xn, uw_ref[...])


def _ffn_gu(xn, gw, uw, chunk=768):
    s, d = xn.shape
    ff = gw.shape[0]
    nsteps = ff // chunk
    sds = jax.ShapeDtypeStruct((s, ff), jnp.float32)
    return pl.pallas_call(
        _ffn_gu_body,
        grid=(nsteps,),
        in_specs=[
            pl.BlockSpec((s, d), lambda k: (0, 0)),
            pl.BlockSpec((chunk, d), lambda k: (k, 0)),
            pl.BlockSpec((chunk, d), lambda k: (k, 0)),
        ],
        out_specs=(pl.BlockSpec((s, chunk), lambda k: (0, k)),
                   pl.BlockSpec((s, chunk), lambda k: (0, k))),
        out_shape=(sds, sds),
        compiler_params=_CP,
    )(xn, gw, uw)


def _ffn_down_body(nsteps, h_ref, y_ref, w_ref, dw_ref, o_ref, acc_ref):
    k = pl.program_id(0)
    part = _mm1x(h_ref[...], dw_ref[...])

    @pl.when(k == 0)
    def _():
        acc_ref[...] = part

    @pl.when(k > 0)
    def _():
        acc_ref[...] += part

    @pl.when(k == nsteps - 1)
    def _():
        o_ref[...] = y_ref[...] + w_ref[...] * acc_ref[...]


def _ffn_down(h, y, wv, dw, chunk=768):
    s, d = y.shape
    ff = h.shape[1]
    nsteps = ff // chunk
    return pl.pallas_call(
        functools.partial(_ffn_down_body, nsteps),
        grid=(nsteps,),
        in_specs=[
            pl.BlockSpec((s, chunk), lambda k: (0, k)),
            pl.BlockSpec((s, d), lambda k: (0, 0)),
            pl.BlockSpec((s, 1), lambda k: (0, 0)),
            pl.BlockSpec((d, chunk), lambda k: (0, k)),
        ],
        out_specs=pl.BlockSpec((s, d), lambda k: (0, 0)),
        out_shape=jax.ShapeDtypeStruct((s, d), jnp.float32),
        scratch_shapes=[pltpu.VMEM((s, d), jnp.float32)],
        compiler_params=_CP,
    )(h, y, wv, dw)


def _head_body(x_ref, nw_ref, nb_ref, w_ref, o_ref, xn_ref):
    @pl.when(pl.program_id(0) == 0)
    def _():
        xn_ref[...] = _ln(x_ref[...], nw_ref[...], nb_ref[...])

    o_ref[...] = _mm1x(xn_ref[...], w_ref[...])


def _lm_head(x, nw, nb, w, chunk=1280):
    s, d = x.shape
    v = w.shape[0]
    nsteps = v // chunk
    return pl.pallas_call(
        _head_body,
        grid=(nsteps,),
        in_specs=[
            pl.BlockSpec((s, d), lambda j: (0, 0)),
            pl.BlockSpec((1, d), lambda j: (0, 0)),
            pl.BlockSpec((1, d), lambda j: (0, 0)),
            pl.BlockSpec((chunk, d), lambda j: (j, 0)),
        ],
        out_specs=pl.BlockSpec((s, chunk), lambda j: (0, j)),
        out_shape=jax.ShapeDtypeStruct((s, v), jnp.float32),
        scratch_shapes=[pltpu.VMEM((s, d), jnp.float32)],
        compiler_params=_CP,
    )(x, nw.reshape(1, d), nb.reshape(1, d), w)


def kernel(input_ids, embedding, pos_emb, router_w, attn_norm_w, attn_norm_b,
           qw, kw, vw, ow, ffn_norm_w, ffn_norm_b, gatew, upw, downw,
           final_norm_w, final_norm_b, lm_head_w):
    b, s = input_ids.shape
    cap = s // 2
    scale = _HD ** -0.5
    nlayers = router_w.shape[0]
    ids = input_ids.reshape(s).astype(jnp.int32)

    x = _embed(ids, embedding, pos_emb[:s])
    causal = jnp.triu(jnp.full((s, s), -1e9, dtype=jnp.float32), k=1)
    for l in range(nlayers):
        wv = _pre(x, router_w[l], cap)
        xn = _ln(x, attn_norm_w[l], attn_norm_b[l])
        q, k, v = _qkv(xn, qw[l], kw[l], vw[l])
        qh = q.reshape(1, s, _H, _HD).transpose(0, 2, 1, 3)
        kh = k.reshape(1, s, _H, _HD).transpose(0, 2, 1, 3)
        vh = v.reshape(1, s, _H, _HD).transpose(0, 2, 1, 3)
        scores = jnp.einsum('bhqd,bhkd->bhqk', qh, kh) * scale + causal
        attn = jax.nn.softmax(scores, axis=-1)
        out = jnp.einsum('bhqk,bhkd->bhqd', attn, vh)
        out = out.transpose(0, 2, 1, 3).reshape(s, _H * _HD)
        y = _post(x, out, ow[l], wv)
        xn2 = _ln(y, ffn_norm_w[l], ffn_norm_b[l])
        g, u = _ffn_gu(xn2, gatew[l], upw[l])
        h = jax.nn.silu(g) * u
        x = _ffn_down(h, y, wv, downw[l])
    logits = _lm_head(x, final_norm_w, final_norm_b, lm_head_w)
    return logits.reshape(b, s, -1)


# speedup vs baseline: 1.7421x; 1.7421x over previous
import functools

import jax
import jax.numpy as jnp
from jax.experimental import pallas as pl
from jax.experimental.pallas import tpu as pltpu

_H = 12
_HD = 64

_CP = pltpu.CompilerParams(vmem_limit_bytes=110 * 1024 * 1024)


_EG = 16


def _embed_body(ids_ref, *refs):
    emb_refs = refs[:_EG]
    pos_ref = refs[_EG]
    o_ref = refs[_EG + 1]
    for j in range(_EG):
        o_ref[0, j, :] = emb_refs[j][0, 0, :] + pos_ref[0, j, :]


def _embed(input_ids, embedding, pos_emb):
    s = input_ids.shape[0]
    v, d = embedding.shape
    g = _EG
    mk = lambda j: pl.BlockSpec((1, 1, d), lambda i, ids, j=j: (ids[i * g + j], 0, 0))
    grid_spec = pltpu.PrefetchScalarGridSpec(
        num_scalar_prefetch=1,
        grid=(s // g,),
        in_specs=[mk(j) for j in range(g)] + [
            pl.BlockSpec((1, g, d), lambda i, ids: (i, 0, 0)),
        ],
        out_specs=pl.BlockSpec((1, g, d), lambda i, ids: (i, 0, 0)),
    )
    out = pl.pallas_call(
        _embed_body,
        grid_spec=grid_spec,
        out_shape=jax.ShapeDtypeStruct((s // g, g, d), jnp.float32),
        compiler_params=_CP,
    )(input_ids, *([embedding.reshape(v, 1, d)] * g),
      pos_emb.reshape(s // g, g, d))
    return out.reshape(s, d)


def _mm1x(a, b, dims=(((1,), (1,)), ((), ()))):
    return jax.lax.dot_general(a.astype(jnp.bfloat16), b.astype(jnp.bfloat16),
                               dims, preferred_element_type=jnp.float32)


def _ln(x, w, b):
    m = jnp.mean(x, axis=-1, keepdims=True)
    v = jnp.mean((x - m) ** 2, axis=-1, keepdims=True)
    return (x - m) / jnp.sqrt(v + 1e-6) * w + b


def _pre_body(cap, x_ref, rw_ref, w_ref):
    x = x_ref[...]
    s = x.shape[0]
    rl = _mm1x(x, rw_ref[...])[:, 0:1]
    bits = jax.lax.bitcast_convert_type(rl, jnp.int32)
    skey = jnp.where(bits >= 0, bits, bits ^ jnp.int32(0x7FFFFFFF))
    ukey = jax.lax.bitcast_convert_type(skey ^ jnp.int32(-2147483648),
                                        jnp.uint32)

    def step(i, t):
        bit = jnp.uint32(31) - i.astype(jnp.uint32)
        cand = t | (jnp.uint32(1) << bit)
        cnt = jnp.sum((ukey >= cand).astype(jnp.float32))
        return jnp.where(cnt >= cap, cand, t)

    t = jax.lax.fori_loop(0, 32, step, jnp.uint32(0))
    gt = ukey > t
    eq = ukey == t
    n_gt = jnp.sum(gt.astype(jnp.float32))
    n_eq_needed = cap - n_gt
    rows = jax.lax.broadcasted_iota(jnp.int32, (s, s), 0)
    cols = jax.lax.broadcasted_iota(jnp.int32, (s, s), 1)
    lower = (rows >= cols).astype(jnp.float32)
    cum_eq = jnp.dot(lower, eq.astype(jnp.float32),
                     preferred_element_type=jnp.float32)
    mask = jnp.where(gt | (eq & (cum_eq <= n_eq_needed)), 1.0, 0.0)
    probs = jax.nn.sigmoid(rl)
    w_ref[...] = (mask + probs) - probs


def _pre(x, rw, cap):
    s, d = x.shape
    return pl.pallas_call(
        functools.partial(_pre_body, float(cap)),
        out_shape=jax.ShapeDtypeStruct((s, 1), jnp.float32),
        compiler_params=_CP,
    )(x, jnp.zeros((128, d), jnp.float32).at[0].set(rw))


def _qkv_body(xn_ref, qw_ref, kw_ref, vw_ref, q_ref, k_ref, v_ref):
    xn = xn_ref[...]
    q_ref[...] = _mm1x(xn, qw_ref[...])
    k_ref[...] = _mm1x(xn, kw_ref[...])
    v_ref[...] = _mm1x(xn, vw_ref[...])


def _qkv(xn, qw, kw, vw):
    s, d = xn.shape
    sds = jax.ShapeDtypeStruct((s, d), jnp.float32)
    return pl.pallas_call(
        _qkv_body,
        out_shape=(sds, sds, sds),
        compiler_params=_CP,
    )(xn, qw, kw, vw)


def _post_body(x_ref, a_ref, ow_ref, w_ref, y_ref):
    proj = _mm1x(a_ref[...], ow_ref[...])
    y_ref[...] = x_ref[...] + w_ref[...] * proj


def _post(x, attn_out, ow, wv):
    s, d = x.shape
    return pl.pallas_call(
        _post_body,
        out_shape=jax.ShapeDtypeStruct((s, d), jnp.float32),
        compiler_params=_CP,
    )(x, attn_out, ow, wv)


def _ffn_gu_body(xn_ref, gw_ref, uw_ref, g_ref, u_ref):
    xn = xn_ref[...]
    g_ref[...] = _mm1x(xn, gw_ref[...])
    u_ref[...] = _mm1x(xn, uw_ref[...])


def _ffn_gu(xn, gw, uw, chunk=768):
    s, d = xn.shape
    ff = gw.shape[0]
    nsteps = ff // chunk
    sds = jax.ShapeDtypeStruct((s, ff), jnp.float32)
    return pl.pallas_call(
        _ffn_gu_body,
        grid=(nsteps,),
        in_specs=[
            pl.BlockSpec((s, d), lambda k: (0, 0)),
            pl.BlockSpec((chunk, d), lambda k: (k, 0)),
            pl.BlockSpec((chunk, d), lambda k: (k, 0)),
        ],
        out_specs=(pl.BlockSpec((s, chunk), lambda k: (0, k)),
                   pl.BlockSpec((s, chunk), lambda k: (0, k))),
        out_shape=(sds, sds),
        compiler_params=_CP,
    )(xn, gw, uw)


def _ffn_down_body(nsteps, h_ref, y_ref, w_ref, dw_ref, o_ref, acc_ref):
    k = pl.program_id(0)
    part = _mm1x(h_ref[...], dw_ref[...])

    @pl.when(k == 0)
    def _():
        acc_ref[...] = part

    @pl.when(k > 0)
    def _():
        acc_ref[...] += part

    @pl.when(k == nsteps - 1)
    def _():
        o_ref[...] = y_ref[...] + w_ref[...] * acc_ref[...]


def _ffn_down(h, y, wv, dw, chunk=768):
    s, d = y.shape
    ff = h.shape[1]
    nsteps = ff // chunk
    return pl.pallas_call(
        functools.partial(_ffn_down_body, nsteps),
        grid=(nsteps,),
        in_specs=[
            pl.BlockSpec((s, chunk), lambda k: (0, k)),
            pl.BlockSpec((s, d), lambda k: (0, 0)),
            pl.BlockSpec((s, 1), lambda k: (0, 0)),
            pl.BlockSpec((d, chunk), lambda k: (0, k)),
        ],
        out_specs=pl.BlockSpec((s, d), lambda k: (0, 0)),
        out_shape=jax.ShapeDtypeStruct((s, d), jnp.float32),
        scratch_shapes=[pltpu.VMEM((s, d), jnp.float32)],
        compiler_params=_CP,
    )(h, y, wv, dw)


def _head_body(x_ref, nw_ref, nb_ref, w_ref, o_ref, xn_ref):
    @pl.when(pl.program_id(0) == 0)
    def _():
        xn_ref[...] = _ln(x_ref[...], nw_ref[...], nb_ref[...])

    o_ref[...] = _mm1x(xn_ref[...], w_ref[...])


def _lm_head(x, nw, nb, w, chunk=1280):
    s, d = x.shape
    v = w.shape[0]
    nsteps = v // chunk
    return pl.pallas_call(
        _head_body,
        grid=(nsteps,),
        in_specs=[
            pl.BlockSpec((s, d), lambda j: (0, 0)),
            pl.BlockSpec((1, d), lambda j: (0, 0)),
            pl.BlockSpec((1, d), lambda j: (0, 0)),
            pl.BlockSpec((chunk, d), lambda j: (j, 0)),
        ],
        out_specs=pl.BlockSpec((s, chunk), lambda j: (0, j)),
        out_shape=jax.ShapeDtypeStruct((s, v), jnp.float32),
        scratch_shapes=[pltpu.VMEM((s, d), jnp.float32)],
        compiler_params=_CP,
    )(x, nw.reshape(1, d), nb.reshape(1, d), w)


def kernel(input_ids, embedding, pos_emb, router_w, attn_norm_w, attn_norm_b,
           qw, kw, vw, ow, ffn_norm_w, ffn_norm_b, gatew, upw, downw,
           final_norm_w, final_norm_b, lm_head_w):
    b, s = input_ids.shape
    cap = s // 2
    scale = _HD ** -0.5
    nlayers = router_w.shape[0]
    ids = input_ids.reshape(s).astype(jnp.int32)

    x = _embed(ids, embedding, pos_emb[:s])
    causal = jnp.triu(jnp.full((s, s), -1e9, dtype=jnp.float32), k=1)
    for l in range(nlayers):
        wv = _pre(x, router_w[l], cap)
        xn = _ln(x, attn_norm_w[l], attn_norm_b[l])
        q, k, v = _qkv(xn, qw[l], kw[l], vw[l])
        qh = q.reshape(1, s, _H, _HD).transpose(0, 2, 1, 3)
        kh = k.reshape(1, s, _H, _HD).transpose(0, 2, 1, 3)
        vh = v.reshape(1, s, _H, _HD).transpose(0, 2, 1, 3)
        scores = jnp.einsum('bhqd,bhkd->bhqk', qh, kh) * scale + causal
        attn = jax.nn.softmax(scores, axis=-1)
        out = jnp.einsum('bhqk,bhkd->bhqd', attn, vh)
        out = out.transpose(0, 2, 1, 3).reshape(s, _H * _HD)
        y = _post(x, out, ow[l], wv)
        xn2 = _ln(y, ffn_norm_w[l], ffn_norm_b[l])
        g, u = _ffn_gu(xn2, gatew[l], upw[l])
        h = jax.nn.silu(g) * u
        x = _ffn_down(h, y, wv, downw[l])
    logits = _lm_head(x, final_norm_w, final_norm_b, lm_head_w)
    return logits.reshape(b, s, -1)
